# baseline (speedup 1.0000x reference)
import functools

import jax
import jax.numpy as jnp
from jax import lax
from jax.experimental import pallas as pl
from jax.experimental.pallas import tpu as pltpu
from jax.experimental.pallas import tpu_sc as plsc

VOCAB = 1000000
D_MODEL = 64

NC = 2
NS = 16
NW = NC * NS

TOKENS = 4096 * 200
CHUNK = 128
N_CHUNKS = TOKENS // (NW * CHUNK)
NBUF = 4
N_GROUPS = N_CHUNKS // NBUF


def _body(tok_hbm, table_hbm, out_hbm, idx_v, rows_v, *gsems):
    wid = lax.axis_index("s") * NC + lax.axis_index("c")
    idx_row0 = wid * N_CHUNKS
    out_row0 = wid * N_CHUNKS * CHUNK

    pltpu.sync_copy(tok_hbm.at[pl.ds(idx_row0, N_CHUNKS)], idx_v)

    def gather(j, b):
        return pltpu.make_async_copy(
            table_hbm.at[idx_v.at[j]], rows_v.at[b], gsems[b])

    for b in range(NBUF):
        gather(b, b).start()

    @pl.loop(0, N_GROUPS - 1)
    def _(g):
        for b in range(NBUF):
            j = g * NBUF + b
            gather(j, b).wait()
            pltpu.sync_copy(rows_v.at[b],
                            out_hbm.at[pl.ds(out_row0 + j * CHUNK, CHUNK)])
            gather(j + NBUF, b).start()

    for b in range(NBUF):
        j = (N_GROUPS - 1) * NBUF + b
        gather(j, b).wait()
        pltpu.sync_copy(rows_v.at[b],
                        out_hbm.at[pl.ds(out_row0 + j * CHUNK, CHUNK)])


@jax.jit
def kernel(tokens, emb_weight):
    tok2d = tokens.reshape(TOKENS // CHUNK, CHUNK).astype(jnp.int32)
    mesh = plsc.VectorSubcoreMesh(core_axis_name="c", subcore_axis_name="s")
    out = pl.kernel(
        _body,
        out_type=jax.ShapeDtypeStruct((TOKENS, D_MODEL), jnp.float32),
        mesh=mesh,
        scratch_types=[
            pltpu.VMEM((N_CHUNKS, CHUNK), jnp.int32),
            pltpu.VMEM((NBUF, CHUNK, D_MODEL), jnp.float32),
        ] + [pltpu.SemaphoreType.DMA] * NBUF,
        compiler_params=pltpu.CompilerParams(use_tc_tiling_on_sc=False),
    )(tok2d, emb_weight)
    return out.reshape(tokens.shape[0], tokens.shape[1], D_MODEL)


# speedup vs baseline: 1.0016x; 1.0016x over previous
import jax
import jax.numpy as jnp
from jax import lax
from jax.experimental import pallas as pl
from jax.experimental.pallas import tpu as pltpu
from jax.experimental.pallas import tpu_sc as plsc

VOCAB = 1000000
D_MODEL = 64

NC = 2
NS = 16
NW = NC * NS

TOKENS = 4096 * 200
CHUNK = 128
N_CHUNKS = TOKENS // (NW * CHUNK)
NBUF = 8
LEAD = 4


def _body(tok_hbm, table_hbm, out_hbm, idx_v, rows_v, *sems):
    gsems, osems = sems[:NBUF], sems[NBUF:]
    wid = lax.axis_index("s") * NC + lax.axis_index("c")
    idx_row0 = wid * N_CHUNKS
    out_row0 = wid * N_CHUNKS * CHUNK

    pltpu.sync_copy(tok_hbm.at[pl.ds(idx_row0, N_CHUNKS)], idx_v)

    def gather(j, b):
        return pltpu.make_async_copy(
            table_hbm.at[idx_v.at[j]], rows_v.at[b], gsems[b])

    def out(j, b):
        return pltpu.make_async_copy(
            rows_v.at[b], out_hbm.at[pl.ds(out_row0 + j * CHUNK, CHUNK)],
            osems[b])

    def step(j, b, bs, do_wait_out, do_start_gather):
        if do_wait_out:
            out(j - NBUF + LEAD, bs).wait()
        if do_start_gather:
            gather(j + LEAD, bs).start()
        gather(j, b).wait()
        out(j, b).start()

    for l in range(LEAD):
        gather(l, l).start()

    for j in range(NBUF):
        step(j, j, (j + LEAD) % NBUF, j - NBUF + LEAD >= 0, True)

    @pl.loop(1, N_CHUNKS // NBUF - 1)
    def _(g):
        for b in range(NBUF):
            j = g * NBUF + b
            step(j, b, (b + LEAD) % NBUF, True, True)

    for b in range(NBUF):
        j = N_CHUNKS - NBUF + b
        step(j, b, (b + LEAD) % NBUF, True, j + LEAD < N_CHUNKS)

    for b in range(NBUF - LEAD, NBUF):
        out(N_CHUNKS - NBUF + b, b).wait()


@jax.jit
def kernel(tokens, emb_weight):
    tok2d = tokens.reshape(TOKENS // CHUNK, CHUNK).astype(jnp.int32)
    mesh = plsc.VectorSubcoreMesh(core_axis_name="c", subcore_axis_name="s")
    out = pl.kernel(
        _body,
        out_type=jax.ShapeDtypeStruct((TOKENS, D_MODEL), jnp.float32),
        mesh=mesh,
        scratch_types=[
            pltpu.VMEM((N_CHUNKS, CHUNK), jnp.int32),
            pltpu.VMEM((NBUF, CHUNK, D_MODEL), jnp.float32),
        ] + [pltpu.SemaphoreType.DMA] * (2 * NBUF),
        compiler_params=pltpu.CompilerParams(use_tc_tiling_on_sc=False),
    )(tok2d, emb_weight)
    return out.reshape(tokens.shape[0], tokens.shape[1], D_MODEL)
